# baseline (speedup 1.0000x reference)
import functools

import jax
import jax.numpy as jnp
from jax import lax
from jax.experimental import pallas as pl
from jax.experimental.pallas import tpu as pltpu
from jax.experimental.pallas import tpu_sc as plsc

V = 1000000
D = 64
B = 16384
NEG = 5
NC = 2
NS = 16
NW = NC * NS
BPW = B // NW
CHUNK = 128
NCHUNK = BPW // CHUNK
NGRP = CHUNK // 16


def _sc_scores_kernel(pos_u_hbm, pos_v_hbm, neg_vT_hbm, u2_hbm, v2_hbm,
                      pos_out_hbm, neg_outT_hbm,
                      raw_u, raw_v, raw_n, ku, kv, kn, hu, hv, hn,
                      rows_u, rows_v, rows_n, pos_sbuf, neg_sbuf, sem):
    wid = lax.axis_index("s") * NC + lax.axis_index("c")
    base = wid * BPW
    lanes = lax.iota(jnp.int32, 16)

    def chunk_body(c, _):
        start = base + c * CHUNK
        pltpu.sync_copy(pos_u_hbm.at[pl.ds(start, CHUNK)], raw_u)
        pltpu.sync_copy(pos_v_hbm.at[pl.ds(start, CHUNK)], raw_v)
        for j in range(NEG):
            pltpu.sync_copy(neg_vT_hbm.at[pl.ds(j * B + start, CHUNK)],
                            raw_n.at[pl.ds(j * CHUNK, CHUNK)])
        for g in range(NGRP):
            s = pl.ds(g * 16, 16)
            r = raw_u[s]
            ku[s] = r >> 1
            hu[s] = (r & 1) << 6
            r = raw_v[s]
            kv[s] = r >> 1
            hv[s] = (r & 1) << 6
        for g in range(NEG * NGRP):
            s = pl.ds(g * 16, 16)
            r = raw_n[s]
            kn[s] = r >> 1
            hn[s] = (r & 1) << 6
        cps = [
            pltpu.async_copy(u2_hbm.at[ku], rows_u, sem),
            pltpu.async_copy(v2_hbm.at[kv], rows_v, sem),
        ]
        for j in range(NEG):
            cps.append(pltpu.async_copy(
                v2_hbm.at[kn.at[pl.ds(j * CHUNK, CHUNK)]],
                rows_n.at[pl.ds(j * CHUNK, CHUNK)], sem))
        for cp in cps:
            cp.wait()

        def grp_body(g, _):
            s = pl.ds(g * 16, 16)
            rows16 = g * 16 + lanes
            cu0 = hu[s]
            cv0 = hv[s]
            cn0 = [hn[pl.ds(j * CHUNK + g * 16, 16)] for j in range(NEG)]
            nrows = [j * CHUNK + rows16 for j in range(NEG)]

            def d_body(dd, acc):
                u = plsc.load_gather(rows_u, [rows16, cu0 + dd])
                v = plsc.load_gather(rows_v, [rows16, cv0 + dd])
                accp = acc[0] + u * v
                accn = [
                    acc[1 + j] + u * plsc.load_gather(
                        rows_n, [nrows[j], cn0[j] + dd])
                    for j in range(NEG)
                ]
                return (accp, *accn)

            zero = jnp.zeros((16,), jnp.float32)
            acc = lax.fori_loop(0, D, d_body, (zero,) * (1 + NEG))
            pos_sbuf[s] = acc[0]
            for j in range(NEG):
                neg_sbuf[pl.ds(j * CHUNK + g * 16, 16)] = acc[1 + j]
            return _

        lax.fori_loop(0, NGRP, grp_body, 0)
        pltpu.sync_copy(pos_sbuf, pos_out_hbm.at[pl.ds(start, CHUNK)])
        for j in range(NEG):
            pltpu.sync_copy(neg_sbuf.at[pl.ds(j * CHUNK, CHUNK)],
                            neg_outT_hbm.at[pl.ds(j * B + start, CHUNK)])
        return _

    lax.fori_loop(0, NCHUNK, chunk_body, 0)


@jax.jit
def _sc_scores(pos_u, pos_v, neg_vT, u2, v2):
    mesh = plsc.VectorSubcoreMesh(core_axis_name="c", subcore_axis_name="s")
    return pl.kernel(
        _sc_scores_kernel,
        mesh=mesh,
        compiler_params=pltpu.CompilerParams(needs_layout_passes=False),
        out_type=[
            jax.ShapeDtypeStruct((B,), jnp.float32),
            jax.ShapeDtypeStruct((NEG * B,), jnp.float32),
        ],
        scratch_types=[
            pltpu.VMEM((CHUNK,), jnp.int32),
            pltpu.VMEM((CHUNK,), jnp.int32),
            pltpu.VMEM((NEG * CHUNK,), jnp.int32),
            pltpu.VMEM((CHUNK,), jnp.int32),
            pltpu.VMEM((CHUNK,), jnp.int32),
            pltpu.VMEM((NEG * CHUNK,), jnp.int32),
            pltpu.VMEM((CHUNK,), jnp.int32),
            pltpu.VMEM((CHUNK,), jnp.int32),
            pltpu.VMEM((NEG * CHUNK,), jnp.int32),
            pltpu.VMEM((CHUNK, 2 * D), jnp.float32),
            pltpu.VMEM((CHUNK, 2 * D), jnp.float32),
            pltpu.VMEM((NEG * CHUNK, 2 * D), jnp.float32),
            pltpu.VMEM((CHUNK,), jnp.float32),
            pltpu.VMEM((NEG * CHUNK,), jnp.float32),
            pltpu.SemaphoreType.DMA,
        ],
    )(pos_u, pos_v, neg_vT, u2, v2)


def _loss_body(pos_ref, neg_ref, out_ref):
    p = pos_ref[...]
    n = neg_ref[...]
    ls_p = jnp.minimum(p, 0.0) - jnp.log1p(jnp.exp(-jnp.abs(p)))
    ls_n = jnp.minimum(-n, 0.0) - jnp.log1p(jnp.exp(-jnp.abs(n)))
    out_ref[0, 0] = -(jnp.sum(ls_p) + jnp.sum(ls_n))


@jax.jit
def _tc_loss(pos_s, neg_s):
    out = pl.pallas_call(
        _loss_body,
        out_shape=jax.ShapeDtypeStruct((1, 1), jnp.float32),
        out_specs=pl.BlockSpec(memory_space=pltpu.SMEM),
    )(pos_s, neg_s)
    return out[0, 0]


def kernel(pos_u, pos_v, neg_v, u_emb, v_emb):
    pos_u = pos_u.astype(jnp.int32)
    pos_v = pos_v.astype(jnp.int32)
    neg_vT = neg_v.astype(jnp.int32).T.reshape(NEG * B)
    u2 = u_emb.reshape(V // 2, 2 * D)
    v2 = v_emb.reshape(V // 2, 2 * D)
    pos_s, neg_sT = _sc_scores(pos_u, pos_v, neg_vT, u2, v2)
    return _tc_loss(pos_s.reshape(B // 128, 128),
                    neg_sT.reshape(NEG * B // 128, 128))


# speedup vs baseline: 1.1355x; 1.1355x over previous
import functools

import jax
import jax.numpy as jnp
from jax import lax
from jax.experimental import pallas as pl
from jax.experimental.pallas import tpu as pltpu
from jax.experimental.pallas import tpu_sc as plsc

V = 1000000
D = 64
B = 16384
NEG = 5
NC = 2
NS = 16
NW = NC * NS
BPW = B // NW
CHUNK = 128
NCHUNK = BPW // CHUNK
NGRP = CHUNK // 16
DV = D // 16


def _sc_scores_kernel(pos_u_hbm, pos_v_hbm, neg_vT_hbm, u_emb_hbm, v_emb_hbm,
                      pos_out_hbm, neg_outT_hbm,
                      idx_u, idx_v, idx_n,
                      rows_u0, rows_v0, rows_n0,
                      rows_u1, rows_v1, rows_n1,
                      pos_sbuf, neg_sbuf, sem0, sem1):
    wid = lax.axis_index("s") * NC + lax.axis_index("c")
    base = wid * BPW
    lanes = lax.iota(jnp.int32, 16)
    bufs = ((rows_u0, rows_v0, rows_n0, sem0),
            (rows_u1, rows_v1, rows_n1, sem1))

    pltpu.sync_copy(pos_u_hbm.at[pl.ds(base, BPW)], idx_u)
    pltpu.sync_copy(pos_v_hbm.at[pl.ds(base, BPW)], idx_v)
    for j in range(NEG):
        pltpu.sync_copy(neg_vT_hbm.at[pl.ds(j * B + base, BPW)],
                        idx_n.at[pl.ds(j * BPW, BPW)])

    def fire(c, bufset):
        ru, rv, rn, sem = bufset
        cps = [
            pltpu.async_copy(u_emb_hbm.at[idx_u.at[pl.ds(c * CHUNK, CHUNK)]],
                             ru, sem),
            pltpu.async_copy(v_emb_hbm.at[idx_v.at[pl.ds(c * CHUNK, CHUNK)]],
                             rv, sem),
        ]
        for j in range(NEG):
            cps.append(pltpu.async_copy(
                v_emb_hbm.at[idx_n.at[pl.ds(j * BPW + c * CHUNK, CHUNK)]],
                rn.at[pl.ds(j * CHUNK, CHUNK)], sem))
        return cps

    def compute(c, bufset):
        ru, rv, rn, _ = bufset
        start = base + c * CHUNK

        def grp_body(g, _):
            res = [jnp.zeros((16,), jnp.float32) for _ in range(1 + NEG)]
            for ib in range(16):
                b = g * 16 + ib
                lmask = lanes == ib
                us = [ru[b, pl.ds(k * 16, 16)] for k in range(DV)]
                vs = [rv[b, pl.ds(k * 16, 16)] for k in range(DV)]
                pp = sum(u * v for u, v in zip(us, vs))
                res[0] = lax.select(lmask, jnp.full((16,), jnp.sum(pp)),
                                    res[0])
                for j in range(NEG):
                    ns = [rn[j * CHUNK + b, pl.ds(k * 16, 16)]
                          for k in range(DV)]
                    nn = sum(u * nv for u, nv in zip(us, ns))
                    res[1 + j] = lax.select(
                        lmask, jnp.full((16,), jnp.sum(nn)), res[1 + j])
            pos_sbuf[pl.ds(g * 16, 16)] = res[0]
            for j in range(NEG):
                neg_sbuf[pl.ds(j * CHUNK + g * 16, 16)] = res[1 + j]
            return _

        lax.fori_loop(0, NGRP, grp_body, 0)
        pltpu.sync_copy(pos_sbuf, pos_out_hbm.at[pl.ds(start, CHUNK)])
        for j in range(NEG):
            pltpu.sync_copy(neg_sbuf.at[pl.ds(j * CHUNK, CHUNK)],
                            neg_outT_hbm.at[pl.ds(j * B + start, CHUNK)])

    pending = fire(0, bufs[0])
    for c in range(NCHUNK):
        nxt = fire(c + 1, bufs[(c + 1) % 2]) if c + 1 < NCHUNK else None
        for cp in pending:
            cp.wait()
        compute(c, bufs[c % 2])
        pending = nxt


@jax.jit
def _sc_scores(pos_u, pos_v, neg_vT, u_emb, v_emb):
    mesh = plsc.VectorSubcoreMesh(core_axis_name="c", subcore_axis_name="s")
    return pl.kernel(
        _sc_scores_kernel,
        mesh=mesh,
        compiler_params=pltpu.CompilerParams(
            needs_layout_passes=False, use_tc_tiling_on_sc=False),
        out_type=[
            jax.ShapeDtypeStruct((B,), jnp.float32),
            jax.ShapeDtypeStruct((NEG * B,), jnp.float32),
        ],
        scratch_types=[
            pltpu.VMEM((BPW,), jnp.int32),
            pltpu.VMEM((BPW,), jnp.int32),
            pltpu.VMEM((NEG * BPW,), jnp.int32),
            pltpu.VMEM((CHUNK, D), jnp.float32),
            pltpu.VMEM((CHUNK, D), jnp.float32),
            pltpu.VMEM((NEG * CHUNK, D), jnp.float32),
            pltpu.VMEM((CHUNK, D), jnp.float32),
            pltpu.VMEM((CHUNK, D), jnp.float32),
            pltpu.VMEM((NEG * CHUNK, D), jnp.float32),
            pltpu.VMEM((CHUNK,), jnp.float32),
            pltpu.VMEM((NEG * CHUNK,), jnp.float32),
            pltpu.SemaphoreType.DMA,
            pltpu.SemaphoreType.DMA,
        ],
    )(pos_u, pos_v, neg_vT, u_emb, v_emb)


def _loss_body(pos_ref, neg_ref, out_ref):
    p = pos_ref[...]
    n = neg_ref[...]
    ls_p = jnp.minimum(p, 0.0) - jnp.log1p(jnp.exp(-jnp.abs(p)))
    ls_n = jnp.minimum(-n, 0.0) - jnp.log1p(jnp.exp(-jnp.abs(n)))
    out_ref[0, 0] = -(jnp.sum(ls_p) + jnp.sum(ls_n))


@jax.jit
def _tc_loss(pos_s, neg_s):
    out = pl.pallas_call(
        _loss_body,
        out_shape=jax.ShapeDtypeStruct((1, 1), jnp.float32),
        out_specs=pl.BlockSpec(memory_space=pltpu.SMEM),
    )(pos_s, neg_s)
    return out[0, 0]


def kernel(pos_u, pos_v, neg_v, u_emb, v_emb):
    pos_u = pos_u.astype(jnp.int32)
    pos_v = pos_v.astype(jnp.int32)
    neg_vT = neg_v.astype(jnp.int32).T.reshape(NEG * B)
    pos_s, neg_sT = _sc_scores(pos_u, pos_v, neg_vT, u_emb, v_emb)
    return _tc_loss(pos_s.reshape(B // 128, 128),
                    neg_sT.reshape(NEG * B // 128, 128))
